# baseline (speedup 1.0000x reference)
import functools

import jax
import jax.numpy as jnp
from jax import lax
from jax.experimental import pallas as pl
from jax.experimental.pallas import tpu as pltpu
from jax.experimental.pallas import tpu_sc as plsc

N = 10000
E = 320000
D = 128
NPAD = 10240
NC, NS = 2, 16
NW = NC * NS
EPW = E // NW
CH = 80
NCH = EPW // CH
RPT = NPAD // NS
ZR = CH


def _sc_aggregate(x, edge_index, zeros):
    mesh = plsc.VectorSubcoreMesh(
        core_axis_name="core", subcore_axis_name="subcore",
        num_cores=NC, num_subcores=NS)

    @functools.partial(
        pl.kernel,
        out_type=(jax.ShapeDtypeStruct((NC, NPAD, D), jnp.float32),
                  jax.ShapeDtypeStruct((NW, N), jnp.float32)),
        mesh=mesh,
        compiler_params=pltpu.CompilerParams(
            use_tc_tiling_on_sc=False, needs_layout_passes=False),
        scratch_types=[
            pltpu.VMEM((CH,), jnp.int32),
            pltpu.VMEM((CH,), jnp.int32),
            pltpu.VMEM((CH,), jnp.int32),
            pltpu.VMEM((CH,), jnp.int32),
            pltpu.VMEM((CH,), jnp.int32),
            pltpu.VMEM((CH,), jnp.int32),
            pltpu.VMEM((CH, D), jnp.float32),
            pltpu.VMEM((CH, D), jnp.float32),
            pltpu.VMEM((N,), jnp.float32),
            pltpu.SemaphoreType.DMA,
            pltpu.SemaphoreType.DMA,
            pltpu.SemaphoreType.DMA,
            pltpu.SemaphoreType.DMA,
            pltpu.SemaphoreType.DMA,
            pltpu.SemaphoreType.DMA,
            pltpu.VMEM_SHARED((NPAD, D), jnp.float32),
        ],
    )
    def agg_kernel(x_hbm, src_hbm, dst_hbm, z_hbm, out_hbm, cnt_hbm,
                   src0, src1, dstb0, dstb1, dstb2, dstb3, rows0, rows1, hist,
                   gsem0, gsem1, isem0, isem1, ssem0, ssem1, acc):
        cid = lax.axis_index("core")
        sid = lax.axis_index("subcore")
        wid = cid * NS + sid
        base = wid * EPW

        @pl.loop(0, RPT, step=ZR)
        def _(r):
            pltpu.sync_copy(z_hbm, acc.at[pl.ds(sid * RPT + r, ZR)])

        @pl.loop(0, N, step=16)
        def _(i):
            hist[pl.ds(i, 16)] = jnp.zeros((16,), jnp.float32)

        plsc.subcore_barrier()

        ones16 = jnp.ones((16,), jnp.float32)

        def idx_load(g, sv, dv, sem):
            pltpu.async_copy(src_hbm.at[pl.ds(base + g * CH, CH)], sv, sem)
            pltpu.async_copy(dst_hbm.at[pl.ds(base + g * CH, CH)], dv, sem)

        def idx_wait(g, sv, dv, sem):
            pltpu.make_async_copy(
                src_hbm.at[pl.ds(base + g * CH, CH)], sv, sem).wait()
            pltpu.make_async_copy(
                dst_hbm.at[pl.ds(base + g * CH, CH)], dv, sem).wait()

        def gather(sv, rows, sem):
            pltpu.async_copy(x_hbm.at[sv], rows, sem)

        def gwait(sv, rows, sem):
            pltpu.make_async_copy(x_hbm.at[sv], rows, sem).wait()

        def scatter(dv, rows, sem):
            pltpu.async_copy(rows, acc.at[dv], sem, add=True)

        def swait(dv, rows, sem):
            pltpu.make_async_copy(rows, acc.at[dv], sem).wait()

        def count(dv):
            @pl.loop(0, CH, step=16)
            def _(k):
                dvec = dv[pl.ds(k, 16)]
                plsc.addupdate_scatter(hist, [dvec], ones16)

        idx_load(0, src0, dstb0, isem0)
        idx_wait(0, src0, dstb0, isem0)
        gather(src0, rows0, gsem0)
        idx_load(1, src1, dstb1, isem1)

        @pl.loop(0, NCH - 1, step=4)
        def _(g):
            gwait(src0, rows0, gsem0)
            scatter(dstb0, rows0, ssem0)
            count(dstb0)
            idx_wait(g + 1, src1, dstb1, isem1)

            @pl.when(g > 0)
            def _():
                swait(dstb3, rows1, ssem1)

            gather(src1, rows1, gsem1)
            idx_load(g + 2, src0, dstb2, isem0)
            gwait(src1, rows1, gsem1)
            swait(dstb0, rows0, ssem0)
            scatter(dstb1, rows1, ssem1)
            count(dstb1)
            idx_wait(g + 2, src0, dstb2, isem0)
            gather(src0, rows0, gsem0)

            @pl.when(g + 3 < NCH)
            def _():
                idx_load(g + 3, src1, dstb3, isem1)

            gwait(src0, rows0, gsem0)
            scatter(dstb2, rows0, ssem0)
            count(dstb2)

            @pl.when(g + 3 < NCH)
            def _():
                idx_wait(g + 3, src1, dstb3, isem1)
                swait(dstb1, rows1, ssem1)
                gather(src1, rows1, gsem1)

                @pl.when(g + 4 < NCH)
                def _():
                    idx_load(g + 4, src0, dstb0, isem0)

                gwait(src1, rows1, gsem1)
                swait(dstb2, rows0, ssem0)
                scatter(dstb3, rows1, ssem1)
                count(dstb3)

                @pl.when(g + 4 < NCH)
                def _():
                    idx_wait(g + 4, src0, dstb0, isem0)
                    gather(src0, rows0, gsem0)

                    @pl.when(g + 5 < NCH)
                    def _():
                        idx_load(g + 5, src1, dstb1, isem1)

        swait(dstb3, rows1, ssem1)
        gwait(src0, rows0, gsem0)
        scatter(dstb0, rows0, ssem0)
        count(dstb0)
        swait(dstb0, rows0, ssem0)

        plsc.subcore_barrier()
        pltpu.sync_copy(acc.at[pl.ds(sid * RPT, RPT)],
                        out_hbm.at[cid, pl.ds(sid * RPT, RPT)])
        pltpu.sync_copy(hist, cnt_hbm.at[wid])

    return agg_kernel(x, edge_index[0], edge_index[1], zeros)


def _tc_right(x, W_r, b_l):
    BR = 2000

    def body(x_ref, wr_ref, b_ref, o_ref):
        o_ref[...] = (
            jnp.dot(x_ref[...], wr_ref[...], preferred_element_type=jnp.float32)
            + b_ref[...]
        )

    return pl.pallas_call(
        body,
        grid=(N // BR,),
        in_specs=[
            pl.BlockSpec((BR, D), lambda i: (i, 0)),
            pl.BlockSpec((D, D), lambda i: (0, 0)),
            pl.BlockSpec((1, D), lambda i: (0, 0)),
        ],
        out_specs=pl.BlockSpec((BR, D), lambda i: (i, 0)),
        out_shape=jax.ShapeDtypeStruct((N, D), jnp.float32),
    )(x, W_r, b_l.reshape(1, D))


def _tc_combine(partials, counts, z, W_l):
    BR = 2000

    def body(p_ref, c_ref, z_ref, wl_ref, o_ref):
        agg = p_ref[0] + p_ref[1]
        cnt = jnp.sum(c_ref[...], axis=1)[:, None]
        mean = agg / jnp.maximum(cnt, 1.0)
        o_ref[...] = (
            jnp.dot(mean, wl_ref[...], preferred_element_type=jnp.float32)
            + z_ref[...]
        )

    return pl.pallas_call(
        body,
        grid=(N // BR,),
        in_specs=[
            pl.BlockSpec((NC, BR, D), lambda i: (0, i, 0)),
            pl.BlockSpec((BR, NW), lambda i: (i, 0)),
            pl.BlockSpec((BR, D), lambda i: (i, 0)),
            pl.BlockSpec((D, D), lambda i: (0, 0)),
        ],
        out_specs=pl.BlockSpec((BR, D), lambda i: (i, 0)),
        out_shape=jax.ShapeDtypeStruct((N, D), jnp.float32),
    )(partials, counts, z, W_l)


def kernel(x, edge_index, W_l, b_l, W_r):
    zeros = jnp.zeros((ZR, D), jnp.float32)
    partials, counts = _sc_aggregate(x, edge_index, zeros)
    z = _tc_right(x, W_r, b_l)
    return _tc_combine(partials, counts.T, z, W_l)


# speedup vs baseline: 1.0940x; 1.0940x over previous
import functools

import jax
import jax.numpy as jnp
from jax import lax
from jax.experimental import pallas as pl
from jax.experimental.pallas import tpu as pltpu
from jax.experimental.pallas import tpu_sc as plsc

N = 10000
E = 320000
D = 128
DC = 16
NPAD = 10240
NC, NS = 2, 16
NW = NC * NS
EPW = E // NW
CH = 80
NCH = EPW // CH
RPT = NPAD // NS
ZR = CH


def _sc_aggregate(x, src, dst):
    mesh = plsc.VectorSubcoreMesh(
        core_axis_name="core", subcore_axis_name="subcore",
        num_cores=NC, num_subcores=NS)

    @functools.partial(
        pl.kernel,
        out_type=(jax.ShapeDtypeStruct((NC, NPAD, D), jnp.float32),
                  jax.ShapeDtypeStruct((NC, NPAD, DC), jnp.float32)),
        mesh=mesh,
        compiler_params=pltpu.CompilerParams(use_tc_tiling_on_sc=False),
        scratch_types=[
            pltpu.VMEM((CH,), jnp.int32),
            pltpu.VMEM((CH,), jnp.int32),
            pltpu.VMEM((CH,), jnp.int32),
            pltpu.VMEM((CH,), jnp.int32),
            pltpu.VMEM((CH, D), jnp.float32),
            pltpu.VMEM((CH, D), jnp.float32),
            pltpu.VMEM((CH, DC), jnp.float32),
            pltpu.SemaphoreType.DMA,
            pltpu.SemaphoreType.DMA,
            pltpu.SemaphoreType.DMA,
            pltpu.SemaphoreType.DMA,
            pltpu.VMEM_SHARED((NPAD, D), jnp.float32),
            pltpu.VMEM_SHARED((NPAD, DC), jnp.float32),
        ],
    )
    def agg_kernel(x_hbm, src_hbm, dst_hbm, out_hbm, cnt_hbm,
                   src0, src1, dst0, dst1, rows0, rows1, ones_v,
                   gsem0, gsem1, isem0, isem1, acc, cacc):
        cid = lax.axis_index("core")
        sid = lax.axis_index("subcore")
        wid = cid * NS + sid
        base = wid * EPW

        @pl.loop(0, ZR)
        def _(i):
            @pl.loop(0, D, step=16)
            def _(j):
                rows0[pl.ds(i, 1), pl.ds(j, 16)] = jnp.zeros(
                    (1, 16), jnp.float32)

        @pl.loop(0, RPT, step=ZR)
        def _(r):
            pltpu.sync_copy(rows0, acc.at[pl.ds(sid * RPT + r, ZR)])

        @pl.loop(0, CH)
        def _(i):
            ones_v[pl.ds(i, 1), pl.ds(0, 16)] = jnp.zeros((1, 16), jnp.float32)

        @pl.loop(0, RPT, step=ZR)
        def _(r):
            pltpu.sync_copy(ones_v, cacc.at[pl.ds(sid * RPT + r, ZR)])

        @pl.loop(0, CH)
        def _(i):
            ones_v[pl.ds(i, 1), pl.ds(0, 16)] = jnp.ones((1, 16), jnp.float32)

        plsc.subcore_barrier()

        def idx_load(g, sv, dv, sem):
            pltpu.async_copy(src_hbm.at[pl.ds(base + g * CH, CH)], sv, sem)
            pltpu.async_copy(dst_hbm.at[pl.ds(base + g * CH, CH)], dv, sem)

        def idx_wait(g, sv, dv, sem):
            pltpu.make_async_copy(
                src_hbm.at[pl.ds(base + g * CH, CH)], sv, sem).wait()
            pltpu.make_async_copy(
                dst_hbm.at[pl.ds(base + g * CH, CH)], dv, sem).wait()

        def gather(sv, rows, sem):
            pltpu.async_copy(x_hbm.at[sv], rows, sem)

        def gwait(sv, rows, sem):
            pltpu.make_async_copy(x_hbm.at[sv], rows, sem).wait()

        def scatter(dv, rows):
            pltpu.sync_copy(ones_v, cacc.at[dv], add=True)
            pltpu.sync_copy(rows, acc.at[dv], add=True)

        idx_load(0, src0, dst0, isem0)
        idx_wait(0, src0, dst0, isem0)
        gather(src0, rows0, gsem0)
        idx_load(1, src1, dst1, isem1)

        @pl.loop(0, NCH - 1, step=2)
        def _(g):
            gwait(src0, rows0, gsem0)
            idx_wait(g + 1, src1, dst1, isem1)
            gather(src1, rows1, gsem1)
            scatter(dst0, rows0)

            @pl.when(g + 2 < NCH)
            def _():
                idx_load(g + 2, src0, dst0, isem0)

            gwait(src1, rows1, gsem1)

            @pl.when(g + 2 < NCH)
            def _():
                idx_wait(g + 2, src0, dst0, isem0)
                gather(src0, rows0, gsem0)

            scatter(dst1, rows1)

            @pl.when(g + 3 < NCH)
            def _():
                idx_load(g + 3, src1, dst1, isem1)

        gwait(src0, rows0, gsem0)
        scatter(dst0, rows0)

        plsc.subcore_barrier()
        pltpu.sync_copy(acc.at[pl.ds(sid * RPT, RPT)],
                        out_hbm.at[cid, pl.ds(sid * RPT, RPT)])
        pltpu.sync_copy(cacc.at[pl.ds(sid * RPT, RPT)],
                        cnt_hbm.at[cid, pl.ds(sid * RPT, RPT)])

    return agg_kernel(x, src, dst)


def _tc_right(x, W_r, b_l):
    BR = 2000

    def body(x_ref, wr_ref, b_ref, o_ref):
        o_ref[...] = (
            jnp.dot(x_ref[...], wr_ref[...], preferred_element_type=jnp.float32)
            + b_ref[...]
        )

    return pl.pallas_call(
        body,
        grid=(N // BR,),
        in_specs=[
            pl.BlockSpec((BR, D), lambda i: (i, 0)),
            pl.BlockSpec((D, D), lambda i: (0, 0)),
            pl.BlockSpec((1, D), lambda i: (0, 0)),
        ],
        out_specs=pl.BlockSpec((BR, D), lambda i: (i, 0)),
        out_shape=jax.ShapeDtypeStruct((N, D), jnp.float32),
    )(x, W_r, b_l.reshape(1, D))


def _tc_combine(partials, counts, z, W_l):
    BR = 2000

    def body(p_ref, c_ref, z_ref, wl_ref, o_ref):
        agg = p_ref[0] + p_ref[1]
        cnt = c_ref[0, :, :1] + c_ref[1, :, :1]
        mean = agg / jnp.maximum(cnt, 1.0)
        o_ref[...] = (
            jnp.dot(mean, wl_ref[...], preferred_element_type=jnp.float32)
            + z_ref[...]
        )

    return pl.pallas_call(
        body,
        grid=(N // BR,),
        in_specs=[
            pl.BlockSpec((NC, BR, D), lambda i: (0, i, 0)),
            pl.BlockSpec((NC, BR, DC), lambda i: (0, i, 0)),
            pl.BlockSpec((BR, D), lambda i: (i, 0)),
            pl.BlockSpec((D, D), lambda i: (0, 0)),
        ],
        out_specs=pl.BlockSpec((BR, D), lambda i: (i, 0)),
        out_shape=jax.ShapeDtypeStruct((N, D), jnp.float32),
    )(partials, counts, z, W_l)


def kernel(x, edge_index, W_l, b_l, W_r):
    src = edge_index[0]
    dst = edge_index[1]
    partials, counts = _sc_aggregate(x, src, dst)
    z = _tc_right(x, W_r, b_l)
    return _tc_combine(partials, counts, z, W_l)


# speedup vs baseline: 1.0986x; 1.0042x over previous
import functools

import jax
import jax.numpy as jnp
from jax import lax
from jax.experimental import pallas as pl
from jax.experimental.pallas import tpu as pltpu
from jax.experimental.pallas import tpu_sc as plsc

N = 10000
E = 320000
D = 128
DC = 16
NPAD = 10240
NC, NS = 2, 16
NW = NC * NS
EPW = E // NW
CH = 80
NCH = EPW // CH
RPT = NPAD // NS
ZR = CH


def _sc_aggregate(x, src, dst):
    mesh = plsc.VectorSubcoreMesh(
        core_axis_name="core", subcore_axis_name="subcore",
        num_cores=NC, num_subcores=NS)

    @functools.partial(
        pl.kernel,
        out_type=(jax.ShapeDtypeStruct((NC, NPAD, D), jnp.float32),
                  jax.ShapeDtypeStruct((NC, NPAD, DC), jnp.float32)),
        mesh=mesh,
        compiler_params=pltpu.CompilerParams(use_tc_tiling_on_sc=False),
        scratch_types=[
            pltpu.VMEM((CH,), jnp.int32),
            pltpu.VMEM((CH,), jnp.int32),
            pltpu.VMEM((CH,), jnp.int32),
            pltpu.VMEM((CH,), jnp.int32),
            pltpu.VMEM((CH, D), jnp.float32),
            pltpu.VMEM((CH, D), jnp.float32),
            pltpu.VMEM((CH, DC), jnp.float32),
            pltpu.SemaphoreType.DMA,
            pltpu.SemaphoreType.DMA,
            pltpu.SemaphoreType.DMA,
            pltpu.SemaphoreType.DMA,
            pltpu.VMEM_SHARED((NPAD, D), jnp.float32),
            pltpu.VMEM_SHARED((NPAD, DC), jnp.float32),
        ],
    )
    def agg_kernel(x_hbm, src_hbm, dst_hbm, out_hbm, cnt_hbm,
                   src0, src1, dst0, dst1, rows0, rows1, ones_v,
                   gsem0, gsem1, isem0, isem1, acc, cacc):
        cid = lax.axis_index("core")
        sid = lax.axis_index("subcore")
        wid = cid * NS + sid
        base = wid * EPW

        @pl.loop(0, ZR)
        def _(i):
            @pl.loop(0, D, step=16)
            def _(j):
                rows0[pl.ds(i, 1), pl.ds(j, 16)] = jnp.zeros(
                    (1, 16), jnp.float32)

        @pl.loop(0, RPT, step=ZR)
        def _(r):
            pltpu.sync_copy(rows0, acc.at[pl.ds(sid * RPT + r, ZR)])

        @pl.loop(0, CH)
        def _(i):
            ones_v[pl.ds(i, 1), pl.ds(0, 16)] = jnp.zeros((1, 16), jnp.float32)

        @pl.loop(0, RPT, step=ZR)
        def _(r):
            pltpu.sync_copy(ones_v, cacc.at[pl.ds(sid * RPT + r, ZR)])

        @pl.loop(0, CH)
        def _(i):
            ones_v[pl.ds(i, 1), pl.ds(0, 16)] = jnp.ones((1, 16), jnp.float32)

        plsc.subcore_barrier()

        def idx_load(g, sv, dv, sem):
            pltpu.async_copy(src_hbm.at[pl.ds(base + g * CH, CH)], sv, sem)
            pltpu.async_copy(dst_hbm.at[pl.ds(base + g * CH, CH)], dv, sem)

        def idx_wait(g, sv, dv, sem):
            pltpu.make_async_copy(
                src_hbm.at[pl.ds(base + g * CH, CH)], sv, sem).wait()
            pltpu.make_async_copy(
                dst_hbm.at[pl.ds(base + g * CH, CH)], dv, sem).wait()

        def gather(sv, rows, sem):
            pltpu.async_copy(x_hbm.at[sv], rows, sem)

        def gwait(sv, rows, sem):
            pltpu.make_async_copy(x_hbm.at[sv], rows, sem).wait()

        def scatter(dv, rows):
            pltpu.sync_copy(ones_v, cacc.at[dv], add=True)
            pltpu.sync_copy(rows, acc.at[dv], add=True)

        idx_load(0, src0, dst0, isem0)
        idx_wait(0, src0, dst0, isem0)
        gather(src0, rows0, gsem0)
        idx_load(1, src1, dst1, isem1)

        @pl.loop(0, NCH - 1, step=2)
        def _(g):
            gwait(src0, rows0, gsem0)
            idx_wait(g + 1, src1, dst1, isem1)
            gather(src1, rows1, gsem1)
            scatter(dst0, rows0)

            @pl.when(g + 2 < NCH)
            def _():
                idx_load(g + 2, src0, dst0, isem0)

            gwait(src1, rows1, gsem1)

            @pl.when(g + 2 < NCH)
            def _():
                idx_wait(g + 2, src0, dst0, isem0)
                gather(src0, rows0, gsem0)

            scatter(dst1, rows1)

            @pl.when(g + 3 < NCH)
            def _():
                idx_load(g + 3, src1, dst1, isem1)

        gwait(src0, rows0, gsem0)
        scatter(dst0, rows0)

        plsc.subcore_barrier()
        pltpu.sync_copy(acc.at[pl.ds(sid * RPT, RPT)],
                        out_hbm.at[cid, pl.ds(sid * RPT, RPT)])
        pltpu.sync_copy(cacc.at[pl.ds(sid * RPT, RPT)],
                        cnt_hbm.at[cid, pl.ds(sid * RPT, RPT)])

    return agg_kernel(x, src, dst)


def _tc_right(x, W_r, b_l):
    BR = 2000

    def body(x_ref, wr_ref, b_ref, o_ref):
        o_ref[...] = (
            jnp.dot(x_ref[...], wr_ref[...], preferred_element_type=jnp.float32)
            + b_ref[...]
        )

    return pl.pallas_call(
        body,
        grid=(N // BR,),
        in_specs=[
            pl.BlockSpec((BR, D), lambda i: (i, 0)),
            pl.BlockSpec((D, D), lambda i: (0, 0)),
            pl.BlockSpec((1, D), lambda i: (0, 0)),
        ],
        out_specs=pl.BlockSpec((BR, D), lambda i: (i, 0)),
        out_shape=jax.ShapeDtypeStruct((N, D), jnp.float32),
    )(x, W_r, b_l.reshape(1, D))


def _tc_combine(partials, counts, x, W_l, b_l, W_r):
    BR = 2000

    def body(p_ref, c_ref, x_ref, wl_ref, wr_ref, b_ref, o_ref):
        agg = p_ref[0] + p_ref[1]
        cnt = c_ref[0, :, :1] + c_ref[1, :, :1]
        mean = agg / jnp.maximum(cnt, 1.0)
        o_ref[...] = (
            jnp.dot(mean, wl_ref[...], preferred_element_type=jnp.float32)
            + jnp.dot(x_ref[...], wr_ref[...], preferred_element_type=jnp.float32)
            + b_ref[...]
        )

    return pl.pallas_call(
        body,
        grid=(N // BR,),
        in_specs=[
            pl.BlockSpec((NC, BR, D), lambda i: (0, i, 0)),
            pl.BlockSpec((NC, BR, DC), lambda i: (0, i, 0)),
            pl.BlockSpec((BR, D), lambda i: (i, 0)),
            pl.BlockSpec((D, D), lambda i: (0, 0)),
            pl.BlockSpec((D, D), lambda i: (0, 0)),
            pl.BlockSpec((1, D), lambda i: (0, 0)),
        ],
        out_specs=pl.BlockSpec((BR, D), lambda i: (i, 0)),
        out_shape=jax.ShapeDtypeStruct((N, D), jnp.float32),
    )(partials, counts, x, W_l, W_r, b_l.reshape(1, D))


def kernel(x, edge_index, W_l, b_l, W_r):
    src = edge_index[0]
    dst = edge_index[1]
    partials, counts = _sc_aggregate(x, src, dst)
    return _tc_combine(partials, counts, x, W_l, b_l, W_r)


# speedup vs baseline: 1.1005x; 1.0017x over previous
import functools

import jax
import jax.numpy as jnp
from jax import lax
from jax.experimental import pallas as pl
from jax.experimental.pallas import tpu as pltpu
from jax.experimental.pallas import tpu_sc as plsc

N = 10000
E = 320000
D = 128
DC = 16
NPAD = 10240
NC, NS = 2, 16
NW = NC * NS
EPW = E // NW
CH = 80
NCH = EPW // CH
RPT = NPAD // NS
ZR = CH


def _sc_aggregate(x, src, dst):
    mesh = plsc.VectorSubcoreMesh(
        core_axis_name="core", subcore_axis_name="subcore",
        num_cores=NC, num_subcores=NS)

    @functools.partial(
        pl.kernel,
        out_type=(jax.ShapeDtypeStruct((NC, NPAD, D), jnp.float32),
                  jax.ShapeDtypeStruct((NC, NPAD, DC), jnp.float32)),
        mesh=mesh,
        compiler_params=pltpu.CompilerParams(use_tc_tiling_on_sc=False),
        scratch_types=[
            pltpu.VMEM((CH,), jnp.int32),
            pltpu.VMEM((CH,), jnp.int32),
            pltpu.VMEM((CH,), jnp.int32),
            pltpu.VMEM((CH,), jnp.int32),
            pltpu.VMEM((CH, D), jnp.float32),
            pltpu.VMEM((CH, D), jnp.float32),
            pltpu.VMEM((CH, DC), jnp.float32),
            pltpu.SemaphoreType.DMA,
            pltpu.SemaphoreType.DMA,
            pltpu.SemaphoreType.DMA,
            pltpu.SemaphoreType.DMA,
            pltpu.SemaphoreType.DMA,
            pltpu.VMEM_SHARED((NPAD, D), jnp.float32),
            pltpu.VMEM_SHARED((NPAD, DC), jnp.float32),
        ],
    )
    def agg_kernel(x_hbm, src_hbm, dst_hbm, out_hbm, cnt_hbm,
                   src0, src1, dst0, dst1, rows0, rows1, ones_v,
                   gsem0, gsem1, isem0, isem1, csem, acc, cacc):
        cid = lax.axis_index("core")
        sid = lax.axis_index("subcore")
        wid = cid * NS + sid
        base = wid * EPW

        @pl.loop(0, ZR)
        def _(i):
            @pl.loop(0, D, step=16)
            def _(j):
                rows0[pl.ds(i, 1), pl.ds(j, 16)] = jnp.zeros(
                    (1, 16), jnp.float32)

        @pl.loop(0, RPT, step=ZR)
        def _(r):
            pltpu.sync_copy(rows0, acc.at[pl.ds(sid * RPT + r, ZR)])

        @pl.loop(0, CH)
        def _(i):
            ones_v[pl.ds(i, 1), pl.ds(0, 16)] = jnp.zeros((1, 16), jnp.float32)

        @pl.loop(0, RPT, step=ZR)
        def _(r):
            pltpu.sync_copy(ones_v, cacc.at[pl.ds(sid * RPT + r, ZR)])

        @pl.loop(0, CH)
        def _(i):
            ones_v[pl.ds(i, 1), pl.ds(0, 16)] = jnp.ones((1, 16), jnp.float32)

        plsc.subcore_barrier()

        def idx_load(g, sv, dv, sem):
            pltpu.async_copy(src_hbm.at[pl.ds(base + g * CH, CH)], sv, sem)
            pltpu.async_copy(dst_hbm.at[pl.ds(base + g * CH, CH)], dv, sem)

        def idx_wait(g, sv, dv, sem):
            pltpu.make_async_copy(
                src_hbm.at[pl.ds(base + g * CH, CH)], sv, sem).wait()
            pltpu.make_async_copy(
                dst_hbm.at[pl.ds(base + g * CH, CH)], dv, sem).wait()

        def gather(sv, rows, sem):
            pltpu.async_copy(x_hbm.at[sv], rows, sem)

        def gwait(sv, rows, sem):
            pltpu.make_async_copy(x_hbm.at[sv], rows, sem).wait()

        def scatter(dv, rows):
            pltpu.async_copy(ones_v, cacc.at[dv], csem, add=True)
            pltpu.sync_copy(rows, acc.at[dv], add=True)
            pltpu.make_async_copy(ones_v, cacc.at[dv], csem).wait()

        idx_load(0, src0, dst0, isem0)
        idx_wait(0, src0, dst0, isem0)
        gather(src0, rows0, gsem0)
        idx_load(1, src1, dst1, isem1)

        @pl.loop(0, NCH - 1, step=2)
        def _(g):
            gwait(src0, rows0, gsem0)
            idx_wait(g + 1, src1, dst1, isem1)
            gather(src1, rows1, gsem1)
            scatter(dst0, rows0)

            @pl.when(g + 2 < NCH)
            def _():
                idx_load(g + 2, src0, dst0, isem0)

            gwait(src1, rows1, gsem1)

            @pl.when(g + 2 < NCH)
            def _():
                idx_wait(g + 2, src0, dst0, isem0)
                gather(src0, rows0, gsem0)

            scatter(dst1, rows1)

            @pl.when(g + 3 < NCH)
            def _():
                idx_load(g + 3, src1, dst1, isem1)

        gwait(src0, rows0, gsem0)
        scatter(dst0, rows0)

        plsc.subcore_barrier()
        pltpu.sync_copy(acc.at[pl.ds(sid * RPT, RPT)],
                        out_hbm.at[cid, pl.ds(sid * RPT, RPT)])
        pltpu.sync_copy(cacc.at[pl.ds(sid * RPT, RPT)],
                        cnt_hbm.at[cid, pl.ds(sid * RPT, RPT)])

    return agg_kernel(x, src, dst)


def _tc_right(x, W_r, b_l):
    BR = 2000

    def body(x_ref, wr_ref, b_ref, o_ref):
        o_ref[...] = (
            jnp.dot(x_ref[...], wr_ref[...], preferred_element_type=jnp.float32)
            + b_ref[...]
        )

    return pl.pallas_call(
        body,
        grid=(N // BR,),
        in_specs=[
            pl.BlockSpec((BR, D), lambda i: (i, 0)),
            pl.BlockSpec((D, D), lambda i: (0, 0)),
            pl.BlockSpec((1, D), lambda i: (0, 0)),
        ],
        out_specs=pl.BlockSpec((BR, D), lambda i: (i, 0)),
        out_shape=jax.ShapeDtypeStruct((N, D), jnp.float32),
    )(x, W_r, b_l.reshape(1, D))


def _tc_combine(partials, counts, x, W_l, b_l, W_r):
    BR = 2000

    def body(p_ref, c_ref, x_ref, wl_ref, wr_ref, b_ref, o_ref):
        agg = p_ref[0] + p_ref[1]
        cnt = c_ref[0, :, :1] + c_ref[1, :, :1]
        mean = agg / jnp.maximum(cnt, 1.0)
        o_ref[...] = (
            jnp.dot(mean, wl_ref[...], preferred_element_type=jnp.float32)
            + jnp.dot(x_ref[...], wr_ref[...], preferred_element_type=jnp.float32)
            + b_ref[...]
        )

    return pl.pallas_call(
        body,
        grid=(N // BR,),
        in_specs=[
            pl.BlockSpec((NC, BR, D), lambda i: (0, i, 0)),
            pl.BlockSpec((NC, BR, DC), lambda i: (0, i, 0)),
            pl.BlockSpec((BR, D), lambda i: (i, 0)),
            pl.BlockSpec((D, D), lambda i: (0, 0)),
            pl.BlockSpec((D, D), lambda i: (0, 0)),
            pl.BlockSpec((1, D), lambda i: (0, 0)),
        ],
        out_specs=pl.BlockSpec((BR, D), lambda i: (i, 0)),
        out_shape=jax.ShapeDtypeStruct((N, D), jnp.float32),
    )(partials, counts, x, W_l, W_r, b_l.reshape(1, D))


def kernel(x, edge_index, W_l, b_l, W_r):
    src = edge_index[0]
    dst = edge_index[1]
    partials, counts = _sc_aggregate(x, src, dst)
    return _tc_combine(partials, counts, x, W_l, b_l, W_r)
